# Initial kernel scaffold; baseline (speedup 1.0000x reference)
import jax
import jax.numpy as jnp
from jax.experimental import pallas as pl


def kernel(x):
    raise NotImplementedError("write your pallas kernel here")


# speedup vs baseline: 1.3064x; 1.3064x over previous
import jax
import jax.numpy as jnp
from jax import lax
from jax.experimental import pallas as pl
from jax.experimental.pallas import tpu as pltpu
from jax.experimental.pallas import tpu_sc as plsc

NCOLS = 26
CARD = 100
NBATCH = 16384
ROW = NCOLS * CARD
NWORKERS = 32
ROWS_PER_W = NBATCH // NWORKERS
CHUNK_ROWS = 16
NCHUNKS = ROWS_PER_W // CHUNK_ROWS
CHUNK_WORDS = CHUNK_ROWS * ROW
XW_WORDS = ROWS_PER_W * NCOLS


def _onehot_body(x_hbm, out_hbm, x_v, buf0, buf1, sem0, sem1):
    wid = lax.axis_index("s") * 2 + lax.axis_index("c")
    lanes = lax.iota(jnp.int32, 16)
    ones = jnp.full((16,), 1, jnp.int32)
    zeros = jnp.zeros((16,), jnp.int32)

    pltpu.sync_copy(x_hbm.at[pl.ds(wid * XW_WORDS, XW_WORDS)], x_v)

    def zbody(j, carry):
        buf0[pl.ds(j * 16, 16)] = zeros
        buf1[pl.ds(j * 16, 16)] = zeros
        return carry
    lax.fori_loop(0, CHUNK_WORDS // 16, zbody, 0)

    out_base = wid * ROWS_PER_W * ROW
    bufs = (buf0, buf1)
    sems = (sem0, sem1)

    def mark(cc, buf, val):
        def body(i, carry):
            xv = plsc.load_gather(x_v, [(cc * CHUNK_ROWS + lanes) * NCOLS + i])
            off = lanes * ROW + i * CARD + xv
            plsc.store_scatter(buf, [off], val)
            return carry
        lax.fori_loop(0, NCOLS, body, 0)

    def start(cc, buf, sem):
        pltpu.async_copy(
            buf, out_hbm.at[pl.ds(out_base + cc * CHUNK_WORDS, CHUNK_WORDS)], sem)

    def wait(cc, buf, sem):
        pltpu.make_async_copy(
            buf, out_hbm.at[pl.ds(out_base + cc * CHUNK_WORDS, CHUNK_WORDS)], sem
        ).wait()

    for b in range(2):
        mark(jnp.int32(b), bufs[b], ones)
        start(jnp.int32(b), bufs[b], sems[b])

    def pair_body(p, carry):
        for b in range(2):
            cc = p * 2 + b
            wait(cc - 2, bufs[b], sems[b])
            mark(cc - 2, bufs[b], zeros)
            mark(cc, bufs[b], ones)
            start(cc, bufs[b], sems[b])
        return carry
    lax.fori_loop(1, NCHUNKS // 2, pair_body, 0)

    for b in range(2):
        wait(jnp.int32(NCHUNKS - 2 + b), bufs[b], sems[b])


def kernel(x):
    xf = x.reshape(-1)
    mesh = plsc.VectorSubcoreMesh(core_axis_name="c", subcore_axis_name="s")
    out = pl.kernel(
        _onehot_body,
        out_type=jax.ShapeDtypeStruct((NBATCH * ROW,), jnp.int32),
        mesh=mesh,
        compiler_params=pltpu.CompilerParams(needs_layout_passes=False),
        scratch_types=[
            pltpu.VMEM((XW_WORDS,), jnp.int32),
            pltpu.VMEM((CHUNK_WORDS,), jnp.int32),
            pltpu.VMEM((CHUNK_WORDS,), jnp.int32),
            pltpu.SemaphoreType.DMA,
            pltpu.SemaphoreType.DMA,
        ],
    )(xf)
    return out.reshape(NBATCH, ROW)


# speedup vs baseline: 2.1032x; 1.6099x over previous
import jax
import jax.numpy as jnp
from jax import lax
from jax.experimental import pallas as pl
from jax.experimental.pallas import tpu as pltpu
from jax.experimental.pallas import tpu_sc as plsc

NCOLS = 26
CARD = 100
NBATCH = 16384
ROW = NCOLS * CARD
NWORKERS = 32
ROWS_PER_W = NBATCH // NWORKERS
CHUNK_ROWS = 16
NCHUNKS = ROWS_PER_W // CHUNK_ROWS
CHUNK_WORDS = CHUNK_ROWS * ROW
XW_WORDS = ROWS_PER_W * NCOLS


def _onehot_body(x_hbm, out_hbm, x_v, buf0, buf1, sem0, sem1):
    wid = lax.axis_index("s") * 2 + lax.axis_index("c")
    lanes = lax.iota(jnp.int32, 16)
    ones = jnp.full((16,), 1, jnp.int32)
    zeros = jnp.zeros((16,), jnp.int32)

    pltpu.sync_copy(x_hbm.at[pl.ds(wid * XW_WORDS, XW_WORDS)], x_v)

    def zbody(j, carry):
        p = j * 16 + lanes
        r = p // ROW
        c = p % ROW
        plsc.store_scatter(buf0, [r, c], zeros)
        plsc.store_scatter(buf1, [r, c], zeros)
        return carry
    lax.fori_loop(0, CHUNK_WORDS // 16, zbody, 0)

    row0 = wid * ROWS_PER_W
    bufs = (buf0, buf1)
    sems = (sem0, sem1)

    def mark(cc, buf, val):
        def body(i, carry):
            xv = plsc.load_gather(x_v, [(cc * CHUNK_ROWS + lanes) * NCOLS + i])
            plsc.store_scatter(buf, [lanes, i * CARD + xv], val)
            return carry
        lax.fori_loop(0, NCOLS, body, 0)

    def start(cc, buf, sem):
        pltpu.async_copy(
            buf, out_hbm.at[pl.ds(row0 + cc * CHUNK_ROWS, CHUNK_ROWS)], sem)

    def wait(cc, buf, sem):
        pltpu.make_async_copy(
            buf, out_hbm.at[pl.ds(row0 + cc * CHUNK_ROWS, CHUNK_ROWS)], sem
        ).wait()

    for b in range(2):
        mark(jnp.int32(b), bufs[b], ones)
        start(jnp.int32(b), bufs[b], sems[b])

    def pair_body(p, carry):
        for b in range(2):
            cc = p * 2 + b
            wait(cc - 2, bufs[b], sems[b])
            mark(cc - 2, bufs[b], zeros)
            mark(cc, bufs[b], ones)
            start(cc, bufs[b], sems[b])
        return carry
    lax.fori_loop(1, NCHUNKS // 2, pair_body, 0)

    for b in range(2):
        wait(jnp.int32(NCHUNKS - 2 + b), bufs[b], sems[b])


def kernel(x):
    xf = x.reshape(-1)
    mesh = plsc.VectorSubcoreMesh(core_axis_name="c", subcore_axis_name="s")
    out = pl.kernel(
        _onehot_body,
        out_type=jax.ShapeDtypeStruct((NBATCH, ROW), jnp.int32),
        mesh=mesh,
        compiler_params=pltpu.CompilerParams(
            needs_layout_passes=False, use_tc_tiling_on_sc=True),
        scratch_types=[
            pltpu.VMEM((XW_WORDS,), jnp.int32),
            pltpu.VMEM((CHUNK_ROWS, ROW), jnp.int32),
            pltpu.VMEM((CHUNK_ROWS, ROW), jnp.int32),
            pltpu.SemaphoreType.DMA,
            pltpu.SemaphoreType.DMA,
        ],
    )(xf)
    return out


# speedup vs baseline: 6.8505x; 3.2571x over previous
import jax
import jax.numpy as jnp
from jax import lax
from jax.experimental import pallas as pl
from jax.experimental.pallas import tpu as pltpu
from jax.experimental.pallas import tpu_sc as plsc

NCOLS = 26
CARD = 100
NBATCH = 16384
ROW = NCOLS * CARD
NWORKERS = 32
BPW = NBATCH // NWORKERS
LANES = 128
BLK_R = 2 * CARD
NGRP = BPW // LANES
NBLK_R = ROW // BLK_R
NBLK = NGRP * NBLK_R


def _onehot_body(xt_hbm, out_hbm, xt_v, buf0, buf1, sem0, sem1):
    wid = lax.axis_index("s") * 2 + lax.axis_index("c")
    lanes = lax.iota(jnp.int32, 16)
    ones = jnp.full((16,), 1, jnp.int32)
    zeros = jnp.zeros((16,), jnp.int32)

    b0 = wid * BPW
    pltpu.sync_copy(xt_hbm.at[:, pl.ds(b0, BPW)], xt_v)

    def zbody(r, carry):
        for s in range(LANES // 16):
            buf0[r, pl.ds(s * 16, 16)] = zeros
            buf1[r, pl.ds(s * 16, 16)] = zeros
        return carry
    lax.fori_loop(0, BLK_R, zbody, 0)

    bufs = (buf0, buf1)
    sems = (sem0, sem1)

    def mark(t, buf, val):
        g = t // NBLK_R
        k = t % NBLK_R
        def body(j, carry):
            ii = 2 * k + j
            for s in range(LANES // 16):
                xv = xt_v[ii, pl.ds(g * LANES + s * 16, 16)]
                plsc.store_scatter(
                    buf, [j * CARD + xv, s * 16 + lanes], val)
            return carry
        lax.fori_loop(0, 2, body, 0)

    def _dst(t):
        g = t // NBLK_R
        k = t % NBLK_R
        return out_hbm.at[pl.ds(k * BLK_R, BLK_R),
                          pl.ds(b0 + g * LANES, LANES)]

    def start(t, buf, sem):
        pltpu.async_copy(buf, _dst(t), sem)

    def wait(t, buf, sem):
        pltpu.make_async_copy(buf, _dst(t), sem).wait()

    for b in range(2):
        mark(jnp.int32(b), bufs[b], ones)
        start(jnp.int32(b), bufs[b], sems[b])

    def pair_body(p, carry):
        for b in range(2):
            t = p * 2 + b
            wait(t - 2, bufs[b], sems[b])
            mark(t - 2, bufs[b], zeros)
            mark(t, bufs[b], ones)
            start(t, bufs[b], sems[b])
        return carry
    lax.fori_loop(1, NBLK // 2, pair_body, 0)

    for b in range(2):
        wait(jnp.int32(NBLK - 2 + b), bufs[b], sems[b])


def kernel(x):
    xt = x.T
    mesh = plsc.VectorSubcoreMesh(core_axis_name="c", subcore_axis_name="s")
    out_t = pl.kernel(
        _onehot_body,
        out_type=jax.ShapeDtypeStruct((ROW, NBATCH), jnp.int32),
        mesh=mesh,
        compiler_params=pltpu.CompilerParams(
            needs_layout_passes=False, use_tc_tiling_on_sc=True),
        scratch_types=[
            pltpu.VMEM((NCOLS, BPW), jnp.int32),
            pltpu.VMEM((BLK_R, LANES), jnp.int32),
            pltpu.VMEM((BLK_R, LANES), jnp.int32),
            pltpu.SemaphoreType.DMA,
            pltpu.SemaphoreType.DMA,
        ],
    )(xt)
    return out_t.T
